# baseline (speedup 1.0000x reference)
import jax
import jax.numpy as jnp
from jax import lax
from jax.experimental import pallas as pl
from jax.experimental.pallas import tpu as pltpu
from jax.experimental.pallas import tpu_sc as plsc

N_CORES = 2
N_SUBCORES = 16
LANES = 16
N_WORKERS = N_CORES * N_SUBCORES

BATCH = 16384
DIM = 32
B_PER_W = BATCH // N_WORKERS
CHUNK = 128
N_CHUNKS = B_PER_W // CHUNK


def _mf_body(user_hbm, item_hbm, uw_hbm, iw_hbm, out_hbm,
             uidx, iidx, urows, irows, outv, sem):
    wid = lax.axis_index("s") * N_CORES + lax.axis_index("c")
    base = wid * B_PER_W

    pltpu.sync_copy(user_hbm.at[pl.ds(wid * N_CHUNKS, N_CHUNKS)], uidx)
    pltpu.sync_copy(item_hbm.at[pl.ds(wid * N_CHUNKS, N_CHUNKS)], iidx)

    copies = []
    for k in range(N_CHUNKS):
        copies.append(pltpu.async_copy(
            uw_hbm.at[uidx.at[k]], urows.at[pl.ds(k * CHUNK, CHUNK)], sem))
        copies.append(pltpu.async_copy(
            iw_hbm.at[iidx.at[k]], irows.at[pl.ds(k * CHUNK, CHUNK)], sem))
    for c in copies:
        c.wait()

    lane = lax.iota(jnp.int32, LANES)

    def group(g, carry):
        row = g * LANES + lane
        acc = jnp.zeros((LANES,), jnp.float32)
        for d in range(DIM):
            col = lax.bitwise_and(lane + d, DIM - 1)
            u = plsc.load_gather(urows, [row, col])
            v = plsc.load_gather(irows, [row, col])
            acc = acc + u * v
        outv[pl.ds(g * LANES, LANES)] = 1.0 / (1.0 + jnp.exp(-acc))
        return carry

    lax.fori_loop(0, B_PER_W // LANES, group, 0)
    pltpu.sync_copy(outv, out_hbm.at[pl.ds(base, B_PER_W)])


def kernel(user, item, user_emb_weight, item_emb_weight):
    user2 = user.reshape(N_WORKERS * N_CHUNKS, CHUNK)
    item2 = item.reshape(N_WORKERS * N_CHUNKS, CHUNK)
    mesh = plsc.VectorSubcoreMesh(core_axis_name="c", subcore_axis_name="s")
    f = pl.kernel(
        _mf_body,
        out_type=jax.ShapeDtypeStruct((BATCH,), jnp.float32),
        mesh=mesh,
        compiler_params=pltpu.CompilerParams(
            needs_layout_passes=False, use_tc_tiling_on_sc=False),
        scratch_types=[
            pltpu.VMEM((N_CHUNKS, CHUNK), jnp.int32),
            pltpu.VMEM((N_CHUNKS, CHUNK), jnp.int32),
            pltpu.VMEM((B_PER_W, DIM), jnp.float32),
            pltpu.VMEM((B_PER_W, DIM), jnp.float32),
            pltpu.VMEM((B_PER_W,), jnp.float32),
            pltpu.SemaphoreType.DMA,
        ],
    )
    return f(user2, item2, user_emb_weight, item_emb_weight)


# speedup vs baseline: 3.9306x; 3.9306x over previous
import jax
import jax.numpy as jnp
from jax import lax
from jax.experimental import pallas as pl
from jax.experimental.pallas import tpu as pltpu
from jax.experimental.pallas import tpu_sc as plsc

N_CORES = 2
N_SUBCORES = 16
LANES = 16
N_WORKERS = N_CORES * N_SUBCORES

BATCH = 16384
DIM = 32
B_PER_W = BATCH // N_WORKERS
WBLK = 128
NRING = 8
SLACK = 2


def _win_copy(tbl_hbm, win, sem, ridx, slot):
    c0 = pl.multiple_of(lax.bitwise_and(ridx, jnp.int32(-WBLK)), WBLK)
    return pltpu.make_async_copy(
        tbl_hbm.at[:, pl.ds(c0, WBLK)],
        win.at[pl.ds(slot * DIM, DIM), :],
        sem.at[slot],
    )


def _idx_at(idx_ref, k):
    return idx_ref[pl.ds(k, LANES)][0]


def _mf_body(user_hbm, item_hbm, uwt_hbm, iwt_hbm, out_hbm,
             uidx, iidx, uwin, iwin, urows, irows, outv, usem, isem):
    wid = lax.axis_index("s") * N_CORES + lax.axis_index("c")
    base = wid * B_PER_W

    pltpu.sync_copy(user_hbm.at[pl.ds(base, B_PER_W)],
                    uidx.at[pl.ds(0, B_PER_W)])
    pltpu.sync_copy(item_hbm.at[pl.ds(base, B_PER_W)],
                    iidx.at[pl.ds(0, B_PER_W)])

    lane = lax.iota(jnp.int32, LANES)

    for s in range(NRING - SLACK):
        _win_copy(uwt_hbm, uwin, usem, _idx_at(uidx, s), s).start()
        _win_copy(iwt_hbm, iwin, isem, _idx_at(iidx, s), s).start()

    def extract(win, slot, col):
        row_lo = slot * DIM + lane
        cvec = jnp.full((LANES,), col, jnp.int32) + lane * 0
        lo = plsc.load_gather(win, [row_lo, cvec])
        hi = plsc.load_gather(win, [row_lo + LANES, cvec])
        return lo, hi

    def gather_step(k, carry):
        slot = lax.rem(k, NRING)
        ur = _idx_at(uidx, k)
        ir = _idx_at(iidx, k)
        _win_copy(uwt_hbm, uwin, usem, ur, slot).wait()
        _win_copy(iwt_hbm, iwin, isem, ir, slot).wait()
        ulo, uhi = extract(uwin, slot, lax.bitwise_and(ur, WBLK - 1))
        ilo, ihi = extract(iwin, slot, lax.bitwise_and(ir, WBLK - 1))
        urows[pl.ds(k * DIM, LANES)] = ulo
        urows[pl.ds(k * DIM + LANES, LANES)] = uhi
        irows[pl.ds(k * DIM, LANES)] = ilo
        irows[pl.ds(k * DIM + LANES, LANES)] = ihi

        nxt = k + NRING - SLACK
        nslot = lax.rem(nxt, NRING)

        @pl.when(nxt < B_PER_W)
        def _():
            _win_copy(uwt_hbm, uwin, usem, _idx_at(uidx, nxt), nslot).start()
            _win_copy(iwt_hbm, iwin, isem, _idx_at(iidx, nxt), nslot).start()

        return carry

    lax.fori_loop(0, B_PER_W, gather_step, 0)

    def group(g, carry):
        rbase = g * LANES * DIM + lane * DIM
        acc = jnp.zeros((LANES,), jnp.float32)
        for d in range(DIM):
            off = rbase + lax.bitwise_and(lane + d, DIM - 1)
            u = plsc.load_gather(urows, [off])
            v = plsc.load_gather(irows, [off])
            acc = acc + u * v
        outv[pl.ds(g * LANES, LANES)] = 1.0 / (1.0 + jnp.exp(-acc))
        return carry

    lax.fori_loop(0, B_PER_W // LANES, group, 0)
    pltpu.sync_copy(outv, out_hbm.at[pl.ds(base, B_PER_W)])


def kernel(user, item, user_emb_weight, item_emb_weight):
    uwt = user_emb_weight.T
    iwt = item_emb_weight.T
    mesh = plsc.VectorSubcoreMesh(core_axis_name="c", subcore_axis_name="s")
    f = pl.kernel(
        _mf_body,
        out_type=jax.ShapeDtypeStruct((BATCH,), jnp.float32),
        mesh=mesh,
        compiler_params=pltpu.CompilerParams(
            needs_layout_passes=False, use_tc_tiling_on_sc=True),
        scratch_types=[
            pltpu.VMEM((B_PER_W + LANES,), jnp.int32),
            pltpu.VMEM((B_PER_W + LANES,), jnp.int32),
            pltpu.VMEM((NRING * DIM, WBLK), jnp.float32),
            pltpu.VMEM((NRING * DIM, WBLK), jnp.float32),
            pltpu.VMEM((B_PER_W * DIM,), jnp.float32),
            pltpu.VMEM((B_PER_W * DIM,), jnp.float32),
            pltpu.VMEM((B_PER_W,), jnp.float32),
            pltpu.SemaphoreType.DMA((NRING,)),
            pltpu.SemaphoreType.DMA((NRING,)),
        ],
    )
    return f(user, item, uwt, iwt)
